# baseline (speedup 1.0000x reference)
import functools

import jax
import jax.numpy as jnp
from jax import lax
from jax.experimental import pallas as pl
from jax.experimental.pallas import tpu as pltpu
from jax.experimental.pallas import tpu_sc as plsc

N = 10000
D = 128
DOUT = 40
K = 3
NC = 2
NS = 16
NW = NC * NS
CH = 128
NPT = 320
N_PAD = NW * NPT
NPE = N_PAD // NS
E = 320000
NCH = 80
GC = 16
EPW = NCH * CH
E_PAD = NW * EPW

_MESH = plsc.VectorSubcoreMesh(core_axis_name="c", subcore_axis_name="s")
_SC_PARAMS = pltpu.CompilerParams(needs_layout_passes=False)


@functools.partial(
    pl.kernel,
    mesh=_MESH,
    compiler_params=_SC_PARAMS,
    out_type=jax.ShapeDtypeStruct((NC * N_PAD,), jnp.float32),
    scratch_types=[
        pltpu.VMEM((NCH, CH), jnp.int32),
        pltpu.VMEM((NCH, CH), jnp.int32),
        pltpu.VMEM((NCH, CH), jnp.float32),
        pltpu.VMEM((CH,), jnp.float32),
        pltpu.VMEM((NPE,), jnp.float32),
        pltpu.VMEM_SHARED((N_PAD,), jnp.float32),
    ],
)
def _deg_kernel(src_hbm, dst_hbm, w_hbm, deg_hbm,
                src2d, dst2d, w2d, wmbuf, zb, deg_sh):
    c = lax.axis_index("c")
    s = lax.axis_index("s")
    wid = s * NC + c

    def _z(i, _):
        zb[pl.ds(i * 16, 16)] = jnp.zeros((16,), jnp.float32)
        return 0
    lax.fori_loop(0, NPE // 16, _z, 0)
    pltpu.sync_copy(zb, deg_sh.at[pl.ds(s * NPE, NPE)])
    plsc.subcore_barrier()

    pltpu.sync_copy(src_hbm.at[wid], src2d)
    pltpu.sync_copy(dst_hbm.at[wid], dst2d)
    pltpu.sync_copy(w_hbm.at[wid], w2d)

    def _chunk(i, _):
        def _g(g, _):
            sl = pl.ds(g * 16, 16)
            s16 = src2d[i, sl]
            d16 = dst2d[i, sl]
            w16 = w2d[i, sl]
            wmbuf[sl] = jnp.where(s16 != d16, w16, 0.0)
            return 0
        lax.fori_loop(0, CH // 16, _g, 0)
        pltpu.sync_copy(wmbuf, deg_sh.at[src2d.at[i]], add=True)
        return 0
    lax.fori_loop(0, NCH, _chunk, 0)
    plsc.subcore_barrier()

    pltpu.sync_copy(deg_sh.at[pl.ds(s * NPE, NPE)],
                    deg_hbm.at[pl.ds(c * N_PAD + s * NPE, NPE)])


def _dis_body(deg_ref, dis_ref):
    d = deg_ref[0] + deg_ref[1]
    dis_ref[...] = jnp.where(d > 0.0, lax.rsqrt(jnp.maximum(d, 1e-30)), 0.0)


_dis = pl.pallas_call(
    _dis_body,
    out_shape=jax.ShapeDtypeStruct((N_PAD // D, D), jnp.float32),
)


@functools.partial(
    pl.kernel,
    mesh=_MESH,
    compiler_params=_SC_PARAMS,
    out_type=jax.ShapeDtypeStruct((NW, NCH, CH), jnp.float32),
    scratch_types=[
        pltpu.VMEM((NCH, CH), jnp.int32),
        pltpu.VMEM((NCH, CH), jnp.int32),
        pltpu.VMEM((NCH, CH), jnp.float32),
        pltpu.VMEM((NCH, CH), jnp.float32),
        pltpu.VMEM((N_PAD,), jnp.float32),
    ],
)
def _norm_kernel(dis_hbm, src_hbm, dst_hbm, w_hbm, norm_hbm,
                 src2d, dst2d, w2d, nrm2d, dis_v):
    c = lax.axis_index("c")
    s = lax.axis_index("s")
    wid = s * NC + c

    pltpu.sync_copy(dis_hbm, dis_v)
    pltpu.sync_copy(src_hbm.at[wid], src2d)
    pltpu.sync_copy(dst_hbm.at[wid], dst2d)
    pltpu.sync_copy(w_hbm.at[wid], w2d)

    def _chunk(i, _):
        def _g(g, _):
            sl = pl.ds(g * 16, 16)
            s16 = src2d[i, sl]
            d16 = dst2d[i, sl]
            w16 = w2d[i, sl]
            a = plsc.load_gather(dis_v, [s16])
            b = plsc.load_gather(dis_v, [d16])
            wm = jnp.where(s16 != d16, w16, 0.0)
            nrm2d[i, sl] = -(a * wm * b)
            return 0
        lax.fori_loop(0, CH // 16, _g, 0)
        return 0
    lax.fori_loop(0, NCH, _chunk, 0)
    pltpu.sync_copy(nrm2d, norm_hbm.at[wid])


@functools.partial(
    pl.kernel,
    mesh=_MESH,
    compiler_params=_SC_PARAMS,
    out_type=jax.ShapeDtypeStruct((NC, N_PAD, D), jnp.float32),
    scratch_types=[
        pltpu.VMEM((GC, CH), jnp.int32),
        pltpu.VMEM((GC, CH), jnp.int32),
        pltpu.VMEM((GC, CH), jnp.float32),
        pltpu.VMEM((CH, D), jnp.float32),
        pltpu.VMEM((CH, D), jnp.float32),
        pltpu.VMEM((16, D), jnp.float32),
        pltpu.VMEM_SHARED((N_PAD, D), jnp.float32),
        pltpu.SemaphoreType.DMA,
        pltpu.SemaphoreType.DMA,
        pltpu.SemaphoreType.DMA,
        pltpu.SemaphoreType.DMA,
    ],
)
def _prop_kernel(t_hbm, src_hbm, dst_hbm, norm_hbm, part_hbm,
                 src2d, dst2d, nrm2d, rows0, rows1, zrows, acc_sh,
                 gs0, gs1, ss0, ss1):
    c = lax.axis_index("c")
    s = lax.axis_index("s")
    wid = s * NC + c

    def _z(i, _):
        for j in range(D // 16):
            zrows[i, pl.ds(j * 16, 16)] = jnp.zeros((16,), jnp.float32)
        return 0
    lax.fori_loop(0, 16, _z, 0)

    def _zc(j, _):
        pltpu.sync_copy(zrows, acc_sh.at[pl.ds(s * NPE + j * 16, 16)])
        return 0
    lax.fori_loop(0, NPE // 16, _zc, 0)
    plsc.subcore_barrier()

    def _scale(buf, ci):
        cvec = jnp.full((16,), ci, jnp.int32)

        def _row(e8, _):
            e = e8 * 8
            nbs = [
                plsc.load_gather(nrm2d, [cvec, jnp.full((16,), e + r, jnp.int32)])
                for r in range(8)
            ]
            for r in range(8):
                for j in range(D // 16):
                    sl = pl.ds(j * 16, 16)
                    buf[e + r, sl] = buf[e + r, sl] * nbs[r]
            return 0
        lax.fori_loop(0, CH // 8, _row, 0)

    def _group(g, _):
        pltpu.sync_copy(src_hbm.at[wid, pl.ds(g * GC, GC)], src2d)
        pltpu.sync_copy(dst_hbm.at[wid, pl.ds(g * GC, GC)], dst2d)
        pltpu.sync_copy(norm_hbm.at[wid, pl.ds(g * GC, GC)], nrm2d)
        pltpu.async_copy(t_hbm.at[src2d.at[0]], rows0, gs0)

        def _pair(p, _):
            i0 = 2 * p
            i1 = i0 + 1

            @pl.when(p > 0)
            def _():
                pltpu.make_async_copy(
                    rows1, acc_sh.at[dst2d.at[i0 - 1]], ss1).wait()
            pltpu.async_copy(t_hbm.at[src2d.at[i1]], rows1, gs1)
            pltpu.make_async_copy(t_hbm.at[src2d.at[i0]], rows0, gs0).wait()
            _scale(rows0, i0)
            pltpu.async_copy(rows0, acc_sh.at[dst2d.at[i0]], ss0, add=True)
            pltpu.make_async_copy(t_hbm.at[src2d.at[i1]], rows1, gs1).wait()
            _scale(rows1, i1)
            pltpu.async_copy(rows1, acc_sh.at[dst2d.at[i1]], ss1, add=True)

            @pl.when(p < GC // 2 - 1)
            def _():
                pltpu.make_async_copy(
                    rows0, acc_sh.at[dst2d.at[i0]], ss0).wait()
                pltpu.async_copy(t_hbm.at[src2d.at[i0 + 2]], rows0, gs0)
            return 0
        lax.fori_loop(0, GC // 2, _pair, 0)
        pltpu.make_async_copy(rows0, acc_sh.at[dst2d.at[GC - 2]], ss0).wait()
        pltpu.make_async_copy(rows1, acc_sh.at[dst2d.at[GC - 1]], ss1).wait()
        return 0
    lax.fori_loop(0, NCH // GC, _group, 0)
    plsc.subcore_barrier()

    pltpu.sync_copy(acc_sh.at[pl.ds(s * NPE, NPE)],
                    part_hbm.at[c, pl.ds(s * NPE, NPE)])


BN = 2048


def _c1_body(p_ref, h_ref, w_ref, tx1_ref, part_ref):
    tx1 = p_ref[0] + p_ref[1]
    tx1_ref[...] = tx1
    h = h_ref[...]
    part_ref[...] = (
        jnp.dot(h, w_ref[0], preferred_element_type=jnp.float32)
        + jnp.dot(tx1, w_ref[1], preferred_element_type=jnp.float32)
    )


_c1 = pl.pallas_call(
    _c1_body,
    grid=(N_PAD // BN,),
    in_specs=[
        pl.BlockSpec((NC, BN, D), lambda i: (0, i, 0)),
        pl.BlockSpec((BN, D), lambda i: (i, 0)),
        pl.BlockSpec((K, D, D), lambda i: (0, 0, 0)),
    ],
    out_specs=[
        pl.BlockSpec((BN, D), lambda i: (i, 0)),
        pl.BlockSpec((BN, D), lambda i: (i, 0)),
    ],
    out_shape=[
        jax.ShapeDtypeStruct((N_PAD, D), jnp.float32),
        jax.ShapeDtypeStruct((N_PAD, D), jnp.float32),
    ],
)


def _c2_body(p_ref, h_ref, part_ref, w_ref, b_ref, o_ref, *, final):
    tx2 = 2.0 * (p_ref[0] + p_ref[1]) - h_ref[...]
    logits = (
        part_ref[...]
        + jnp.dot(tx2, w_ref[2], preferred_element_type=jnp.float32)
        + b_ref[...]
    )
    if final:
        logits = jnp.maximum(logits, 0.0)
        col = lax.broadcasted_iota(jnp.int32, logits.shape, 1)
        logits = jnp.where(col < DOUT, logits, -jnp.inf)
        m = jnp.max(logits, axis=1, keepdims=True)
        z = jnp.exp(logits - m)
        lse = jnp.log(jnp.sum(z, axis=1, keepdims=True))
        o_ref[...] = logits - m - lse
    else:
        o_ref[...] = jnp.maximum(logits, 0.0)


def _make_c2(final):
    return pl.pallas_call(
        functools.partial(_c2_body, final=final),
        grid=(N_PAD // BN,),
        in_specs=[
            pl.BlockSpec((NC, BN, D), lambda i: (0, i, 0)),
            pl.BlockSpec((BN, D), lambda i: (i, 0)),
            pl.BlockSpec((BN, D), lambda i: (i, 0)),
            pl.BlockSpec((K, D, D), lambda i: (0, 0, 0)),
            pl.BlockSpec((1, D), lambda i: (0, 0)),
        ],
        out_specs=pl.BlockSpec((BN, D), lambda i: (i, 0)),
        out_shape=jax.ShapeDtypeStruct((N_PAD, D), jnp.float32),
    )


_c2_mid = _make_c2(False)
_c2_fin = _make_c2(True)


def kernel(x, edge_index, edge_attr, W1, b1, W2, b2, W3, b3):
    pad = E_PAD - E
    fill = (jnp.arange(pad, dtype=jnp.int32) % N)
    src = jnp.concatenate([edge_index[0], fill]).reshape(NW, NCH, CH)
    dst = jnp.concatenate([edge_index[1], fill]).reshape(NW, NCH, CH)
    w = jnp.concatenate(
        [edge_attr, jnp.zeros((pad,), jnp.float32)]).reshape(NW, NCH, CH)

    degs = _deg_kernel(src, dst, w)
    dis = _dis(degs.reshape(NC, N_PAD // D, D)).reshape(N_PAD)
    norm = _norm_kernel(dis, src, dst, w)

    xp = jnp.zeros((N_PAD, D), jnp.float32).at[:N].set(x)
    W3p = jnp.zeros((K, D, D), jnp.float32).at[:, :, :DOUT].set(W3)
    b3p = jnp.zeros((D,), jnp.float32).at[:DOUT].set(b3)

    h = xp
    for (W, b, c2) in (
        (W1, b1, _c2_mid),
        (W2, b2, _c2_mid),
        (W3p, b3p, _c2_fin),
    ):
        p1 = _prop_kernel(h, src, dst, norm)
        tx1, part = _c1(p1, h, W)
        p2 = _prop_kernel(tx1, src, dst, norm)
        h = c2(p2, h, part, W, b.reshape(1, D))

    return h[:N, :DOUT]


# speedup vs baseline: 1.2252x; 1.2252x over previous
import functools

import jax
import jax.numpy as jnp
from jax import lax
from jax.experimental import pallas as pl
from jax.experimental.pallas import tpu as pltpu
from jax.experimental.pallas import tpu_sc as plsc

N = 10000
D = 128
DOUT = 40
K = 3
NC = 2
NS = 16
NW = NC * NS
CH = 128
NPT = 320
N_PAD = NW * NPT
NPE = N_PAD // NS
E = 320000
NCH = 80
GC = 16
EPW = NCH * CH
E_PAD = NW * EPW

_MESH = plsc.VectorSubcoreMesh(core_axis_name="c", subcore_axis_name="s")
_SC_PARAMS = pltpu.CompilerParams(needs_layout_passes=False)


@functools.partial(
    pl.kernel,
    mesh=_MESH,
    compiler_params=_SC_PARAMS,
    out_type=jax.ShapeDtypeStruct((NC * N_PAD,), jnp.float32),
    scratch_types=[
        pltpu.VMEM((NCH, CH), jnp.int32),
        pltpu.VMEM((NCH, CH), jnp.int32),
        pltpu.VMEM((NCH, CH), jnp.float32),
        pltpu.VMEM((CH,), jnp.float32),
        pltpu.VMEM((NPE,), jnp.float32),
        pltpu.VMEM_SHARED((N_PAD,), jnp.float32),
    ],
)
def _deg_kernel(src_hbm, dst_hbm, w_hbm, deg_hbm,
                src2d, dst2d, w2d, wmbuf, zb, deg_sh):
    c = lax.axis_index("c")
    s = lax.axis_index("s")
    wid = s * NC + c

    def _z(i, _):
        zb[pl.ds(i * 16, 16)] = jnp.zeros((16,), jnp.float32)
        return 0
    lax.fori_loop(0, NPE // 16, _z, 0)
    pltpu.sync_copy(zb, deg_sh.at[pl.ds(s * NPE, NPE)])
    plsc.subcore_barrier()

    pltpu.sync_copy(src_hbm.at[wid], src2d)
    pltpu.sync_copy(dst_hbm.at[wid], dst2d)
    pltpu.sync_copy(w_hbm.at[wid], w2d)

    def _chunk(i, _):
        def _g(g, _):
            sl = pl.ds(g * 16, 16)
            s16 = src2d[i, sl]
            d16 = dst2d[i, sl]
            w16 = w2d[i, sl]
            wmbuf[sl] = jnp.where(s16 != d16, w16, 0.0)
            return 0
        lax.fori_loop(0, CH // 16, _g, 0)
        pltpu.sync_copy(wmbuf, deg_sh.at[src2d.at[i]], add=True)
        return 0
    lax.fori_loop(0, NCH, _chunk, 0)
    plsc.subcore_barrier()

    pltpu.sync_copy(deg_sh.at[pl.ds(s * NPE, NPE)],
                    deg_hbm.at[pl.ds(c * N_PAD + s * NPE, NPE)])


def _dis_body(deg_ref, dis_ref):
    d = deg_ref[0] + deg_ref[1]
    dis_ref[...] = jnp.where(d > 0.0, lax.rsqrt(jnp.maximum(d, 1e-30)), 0.0)


_dis = pl.pallas_call(
    _dis_body,
    out_shape=jax.ShapeDtypeStruct((N_PAD // D, D), jnp.float32),
)


@functools.partial(
    pl.kernel,
    mesh=_MESH,
    compiler_params=_SC_PARAMS,
    out_type=jax.ShapeDtypeStruct((NW, NCH, CH), jnp.float32),
    scratch_types=[
        pltpu.VMEM((NCH, CH), jnp.int32),
        pltpu.VMEM((NCH, CH), jnp.int32),
        pltpu.VMEM((NCH, CH), jnp.float32),
        pltpu.VMEM((NCH, CH), jnp.float32),
        pltpu.VMEM((N_PAD,), jnp.float32),
    ],
)
def _norm_kernel(dis_hbm, src_hbm, dst_hbm, w_hbm, norm_hbm,
                 src2d, dst2d, w2d, nrm2d, dis_v):
    c = lax.axis_index("c")
    s = lax.axis_index("s")
    wid = s * NC + c

    pltpu.sync_copy(dis_hbm, dis_v)
    pltpu.sync_copy(src_hbm.at[wid], src2d)
    pltpu.sync_copy(dst_hbm.at[wid], dst2d)
    pltpu.sync_copy(w_hbm.at[wid], w2d)

    def _chunk(i, _):
        def _g(g, _):
            sl = pl.ds(g * 16, 16)
            s16 = src2d[i, sl]
            d16 = dst2d[i, sl]
            w16 = w2d[i, sl]
            a = plsc.load_gather(dis_v, [s16])
            b = plsc.load_gather(dis_v, [d16])
            wm = jnp.where(s16 != d16, w16, 0.0)
            nrm2d[i, sl] = -(a * wm * b)
            return 0
        lax.fori_loop(0, CH // 16, _g, 0)
        return 0
    lax.fori_loop(0, NCH, _chunk, 0)
    pltpu.sync_copy(nrm2d, norm_hbm.at[wid])


@functools.partial(
    pl.kernel,
    mesh=_MESH,
    compiler_params=_SC_PARAMS,
    out_type=jax.ShapeDtypeStruct((NC, N_PAD, D), jnp.float32),
    scratch_types=[
        pltpu.VMEM((GC, CH), jnp.int32),
        pltpu.VMEM((GC, CH), jnp.int32),
        pltpu.VMEM((GC, CH), jnp.float32),
        pltpu.VMEM((CH, D), jnp.float32),
        pltpu.VMEM((CH, D), jnp.float32),
        pltpu.VMEM((16, D), jnp.float32),
        pltpu.VMEM_SHARED((N_PAD, D), jnp.float32),
        pltpu.SemaphoreType.DMA,
        pltpu.SemaphoreType.DMA,
        pltpu.SemaphoreType.DMA,
        pltpu.SemaphoreType.DMA,
    ],
)
def _prop_kernel(t_hbm, src_hbm, dst_hbm, norm_hbm, part_hbm,
                 src2d, dst2d, nrm2d, rows0, rows1, zrows, acc_sh,
                 gs0, gs1, ss0, ss1):
    c = lax.axis_index("c")
    s = lax.axis_index("s")
    wid = s * NC + c

    def _z(i, _):
        for j in range(D // 16):
            zrows[i, pl.ds(j * 16, 16)] = jnp.zeros((16,), jnp.float32)
        return 0
    lax.fori_loop(0, 16, _z, 0)

    def _zs(j, _):
        pltpu.async_copy(zrows, acc_sh.at[pl.ds(s * NPE + j * 16, 16)], ss0)
        return 0
    lax.fori_loop(0, NPE // 16, _zs, 0)

    def _zw(j, _):
        pltpu.make_async_copy(
            zrows, acc_sh.at[pl.ds(s * NPE + j * 16, 16)], ss0).wait()
        return 0
    lax.fori_loop(0, NPE // 16, _zw, 0)
    plsc.subcore_barrier()

    def _scale(buf, ci, lo, hi):
        cvec = jnp.full((16,), ci, jnp.int32)

        def _row(e4, _):
            e = e4 * 4
            nbs = [
                plsc.load_gather(nrm2d, [cvec, jnp.full((16,), e + r, jnp.int32)])
                for r in range(4)
            ]
            for r in range(4):
                for j in range(D // 16):
                    sl = pl.ds(j * 16, 16)
                    buf[e + r, sl] = buf[e + r, sl] * nbs[r]
            return 0
        lax.fori_loop(lo // 4, hi // 4, _row, 0)

    HALF = CH // 2

    def _group(g, _):
        pltpu.sync_copy(src_hbm.at[wid, pl.ds(g * GC, GC)], src2d)
        pltpu.sync_copy(dst_hbm.at[wid, pl.ds(g * GC, GC)], dst2d)
        pltpu.sync_copy(norm_hbm.at[wid, pl.ds(g * GC, GC)], nrm2d)
        pltpu.async_copy(t_hbm.at[src2d.at[0]], rows0, gs0)
        pltpu.async_copy(t_hbm.at[src2d.at[1]], rows1, gs1)

        def _pair(p, _):
            i0 = 2 * p
            i1 = i0 + 1
            pltpu.make_async_copy(t_hbm.at[src2d.at[i0]], rows0, gs0).wait()
            _scale(rows0, i0, 0, HALF)

            @pl.when(p > 0)
            def _():
                pltpu.make_async_copy(
                    rows1, acc_sh.at[dst2d.at[i0 - 1]], ss1).wait()
                pltpu.async_copy(t_hbm.at[src2d.at[i1]], rows1, gs1)
            _scale(rows0, i0, HALF, CH)
            pltpu.async_copy(rows0, acc_sh.at[dst2d.at[i0]], ss0, add=True)
            pltpu.make_async_copy(t_hbm.at[src2d.at[i1]], rows1, gs1).wait()
            _scale(rows1, i1, 0, HALF)

            @pl.when(p < GC // 2 - 1)
            def _():
                pltpu.make_async_copy(
                    rows0, acc_sh.at[dst2d.at[i0]], ss0).wait()
                pltpu.async_copy(t_hbm.at[src2d.at[i0 + 2]], rows0, gs0)
            _scale(rows1, i1, HALF, CH)
            pltpu.async_copy(rows1, acc_sh.at[dst2d.at[i1]], ss1, add=True)
            return 0
        lax.fori_loop(0, GC // 2, _pair, 0)
        pltpu.make_async_copy(rows0, acc_sh.at[dst2d.at[GC - 2]], ss0).wait()
        pltpu.make_async_copy(rows1, acc_sh.at[dst2d.at[GC - 1]], ss1).wait()
        return 0
    lax.fori_loop(0, NCH // GC, _group, 0)
    plsc.subcore_barrier()

    pltpu.sync_copy(acc_sh.at[pl.ds(s * NPE, NPE)],
                    part_hbm.at[c, pl.ds(s * NPE, NPE)])


BN = 2048


def _c1_body(p_ref, h_ref, w_ref, tx1_ref, part_ref):
    tx1 = p_ref[0] + p_ref[1]
    tx1_ref[...] = tx1
    h = h_ref[...]
    part_ref[...] = (
        jnp.dot(h, w_ref[0], preferred_element_type=jnp.float32)
        + jnp.dot(tx1, w_ref[1], preferred_element_type=jnp.float32)
    )


_c1 = pl.pallas_call(
    _c1_body,
    grid=(N_PAD // BN,),
    in_specs=[
        pl.BlockSpec((NC, BN, D), lambda i: (0, i, 0)),
        pl.BlockSpec((BN, D), lambda i: (i, 0)),
        pl.BlockSpec((K, D, D), lambda i: (0, 0, 0)),
    ],
    out_specs=[
        pl.BlockSpec((BN, D), lambda i: (i, 0)),
        pl.BlockSpec((BN, D), lambda i: (i, 0)),
    ],
    out_shape=[
        jax.ShapeDtypeStruct((N_PAD, D), jnp.float32),
        jax.ShapeDtypeStruct((N_PAD, D), jnp.float32),
    ],
)


def _c2_body(p_ref, h_ref, part_ref, w_ref, b_ref, o_ref, *, final):
    tx2 = 2.0 * (p_ref[0] + p_ref[1]) - h_ref[...]
    logits = (
        part_ref[...]
        + jnp.dot(tx2, w_ref[2], preferred_element_type=jnp.float32)
        + b_ref[...]
    )
    if final:
        logits = jnp.maximum(logits, 0.0)
        col = lax.broadcasted_iota(jnp.int32, logits.shape, 1)
        logits = jnp.where(col < DOUT, logits, -jnp.inf)
        m = jnp.max(logits, axis=1, keepdims=True)
        z = jnp.exp(logits - m)
        lse = jnp.log(jnp.sum(z, axis=1, keepdims=True))
        o_ref[...] = logits - m - lse
    else:
        o_ref[...] = jnp.maximum(logits, 0.0)


def _make_c2(final):
    return pl.pallas_call(
        functools.partial(_c2_body, final=final),
        grid=(N_PAD // BN,),
        in_specs=[
            pl.BlockSpec((NC, BN, D), lambda i: (0, i, 0)),
            pl.BlockSpec((BN, D), lambda i: (i, 0)),
            pl.BlockSpec((BN, D), lambda i: (i, 0)),
            pl.BlockSpec((K, D, D), lambda i: (0, 0, 0)),
            pl.BlockSpec((1, D), lambda i: (0, 0)),
        ],
        out_specs=pl.BlockSpec((BN, D), lambda i: (i, 0)),
        out_shape=jax.ShapeDtypeStruct((N_PAD, D), jnp.float32),
    )


_c2_mid = _make_c2(False)
_c2_fin = _make_c2(True)


def kernel(x, edge_index, edge_attr, W1, b1, W2, b2, W3, b3):
    pad = E_PAD - E
    fill = (jnp.arange(pad, dtype=jnp.int32) % N)
    src = jnp.concatenate([edge_index[0], fill]).reshape(NW, NCH, CH)
    dst = jnp.concatenate([edge_index[1], fill]).reshape(NW, NCH, CH)
    w = jnp.concatenate(
        [edge_attr, jnp.zeros((pad,), jnp.float32)]).reshape(NW, NCH, CH)

    degs = _deg_kernel(src, dst, w)
    dis = _dis(degs.reshape(NC, N_PAD // D, D)).reshape(N_PAD)
    norm = _norm_kernel(dis, src, dst, w)

    xp = jnp.zeros((N_PAD, D), jnp.float32).at[:N].set(x)
    W3p = jnp.zeros((K, D, D), jnp.float32).at[:, :, :DOUT].set(W3)
    b3p = jnp.zeros((D,), jnp.float32).at[:DOUT].set(b3)

    h = xp
    for (W, b, c2) in (
        (W1, b1, _c2_mid),
        (W2, b2, _c2_mid),
        (W3p, b3p, _c2_fin),
    ):
        p1 = _prop_kernel(h, src, dst, norm)
        tx1, part = _c1(p1, h, W)
        p2 = _prop_kernel(tx1, src, dst, norm)
        h = c2(p2, h, part, W, b.reshape(1, D))

    return h[:N, :DOUT]


# speedup vs baseline: 1.3201x; 1.0775x over previous
import functools

import jax
import jax.numpy as jnp
from jax import lax
from jax.experimental import pallas as pl
from jax.experimental.pallas import tpu as pltpu
from jax.experimental.pallas import tpu_sc as plsc

N = 10000
D = 128
DOUT = 40
K = 3
NC = 2
NS = 16
NW = NC * NS
CH = 128
NPT = 320
N_PAD = NW * NPT
NPE = N_PAD // NS
E = 320000
NCH = 80
GC = 16
EPW = NCH * CH
E_PAD = NW * EPW

_MESH = plsc.VectorSubcoreMesh(core_axis_name="c", subcore_axis_name="s")
_SC_PARAMS = pltpu.CompilerParams(needs_layout_passes=False)


@functools.partial(
    pl.kernel,
    mesh=_MESH,
    compiler_params=_SC_PARAMS,
    out_type=jax.ShapeDtypeStruct((NC * N_PAD,), jnp.float32),
    scratch_types=[
        pltpu.VMEM((NCH, CH), jnp.int32),
        pltpu.VMEM((NCH, CH), jnp.int32),
        pltpu.VMEM((NCH, CH), jnp.float32),
        pltpu.VMEM((CH,), jnp.float32),
        pltpu.VMEM((NPE,), jnp.float32),
        pltpu.VMEM_SHARED((N_PAD,), jnp.float32),
    ],
)
def _deg_kernel(src_hbm, dst_hbm, w_hbm, deg_hbm,
                src2d, dst2d, w2d, wmbuf, zb, deg_sh):
    c = lax.axis_index("c")
    s = lax.axis_index("s")
    wid = s * NC + c

    def _z(i, _):
        zb[pl.ds(i * 16, 16)] = jnp.zeros((16,), jnp.float32)
        return 0
    lax.fori_loop(0, NPE // 16, _z, 0)
    pltpu.sync_copy(zb, deg_sh.at[pl.ds(s * NPE, NPE)])
    plsc.subcore_barrier()

    pltpu.sync_copy(src_hbm.at[wid], src2d)
    pltpu.sync_copy(dst_hbm.at[wid], dst2d)
    pltpu.sync_copy(w_hbm.at[wid], w2d)

    def _chunk(i, _):
        def _g(g, _):
            sl = pl.ds(g * 16, 16)
            s16 = src2d[i, sl]
            d16 = dst2d[i, sl]
            w16 = w2d[i, sl]
            wmbuf[sl] = jnp.where(s16 != d16, w16, 0.0)
            return 0
        lax.fori_loop(0, CH // 16, _g, 0)
        pltpu.sync_copy(wmbuf, deg_sh.at[src2d.at[i]], add=True)
        return 0
    lax.fori_loop(0, NCH, _chunk, 0)
    plsc.subcore_barrier()

    pltpu.sync_copy(deg_sh.at[pl.ds(s * NPE, NPE)],
                    deg_hbm.at[pl.ds(c * N_PAD + s * NPE, NPE)])


def _dis_body(deg_ref, dis_ref):
    d = deg_ref[0] + deg_ref[1]
    dis_ref[...] = jnp.where(d > 0.0, lax.rsqrt(jnp.maximum(d, 1e-30)), 0.0)


_dis = pl.pallas_call(
    _dis_body,
    out_shape=jax.ShapeDtypeStruct((N_PAD // D, D), jnp.float32),
)


@functools.partial(
    pl.kernel,
    mesh=_MESH,
    compiler_params=_SC_PARAMS,
    out_type=jax.ShapeDtypeStruct((NW, NCH, CH), jnp.float32),
    scratch_types=[
        pltpu.VMEM((NCH, CH), jnp.int32),
        pltpu.VMEM((NCH, CH), jnp.int32),
        pltpu.VMEM((NCH, CH), jnp.float32),
        pltpu.VMEM((NCH, CH), jnp.float32),
        pltpu.VMEM((N_PAD,), jnp.float32),
    ],
)
def _norm_kernel(dis_hbm, src_hbm, dst_hbm, w_hbm, norm_hbm,
                 src2d, dst2d, w2d, nrm2d, dis_v):
    c = lax.axis_index("c")
    s = lax.axis_index("s")
    wid = s * NC + c

    pltpu.sync_copy(dis_hbm, dis_v)
    pltpu.sync_copy(src_hbm.at[wid], src2d)
    pltpu.sync_copy(dst_hbm.at[wid], dst2d)
    pltpu.sync_copy(w_hbm.at[wid], w2d)

    def _chunk(i, _):
        def _g(g, _):
            sl = pl.ds(g * 16, 16)
            s16 = src2d[i, sl]
            d16 = dst2d[i, sl]
            w16 = w2d[i, sl]
            a = plsc.load_gather(dis_v, [s16])
            b = plsc.load_gather(dis_v, [d16])
            wm = jnp.where(s16 != d16, w16, 0.0)
            nrm2d[i, sl] = -(a * wm * b)
            return 0
        lax.fori_loop(0, CH // 16, _g, 0)
        return 0
    lax.fori_loop(0, NCH, _chunk, 0)
    pltpu.sync_copy(nrm2d, norm_hbm.at[wid])


@functools.partial(
    pl.kernel,
    mesh=_MESH,
    compiler_params=_SC_PARAMS,
    out_type=jax.ShapeDtypeStruct((NC, N_PAD, D), jnp.float32),
    scratch_types=[
        pltpu.VMEM((GC, CH), jnp.int32),
        pltpu.VMEM((GC, CH), jnp.int32),
        pltpu.VMEM((GC, CH), jnp.float32),
        pltpu.VMEM((CH, D), jnp.float32),
        pltpu.VMEM((CH, D), jnp.float32),
        pltpu.VMEM((16, D), jnp.float32),
        pltpu.VMEM_SHARED((N_PAD, D), jnp.float32),
        pltpu.SemaphoreType.DMA,
        pltpu.SemaphoreType.DMA,
        pltpu.SemaphoreType.DMA,
        pltpu.SemaphoreType.DMA,
    ],
)
def _prop_kernel(t_hbm, src_hbm, dst_hbm, norm_hbm, part_hbm,
                 src2d, dst2d, nrm2d, rows0, rows1, zrows, acc_sh,
                 gs0, gs1, ss0, ss1):
    c = lax.axis_index("c")
    s = lax.axis_index("s")
    wid = s * NC + c

    def _z(i, _):
        for j in range(D // 16):
            zrows[i, pl.ds(j * 16, 16)] = jnp.zeros((16,), jnp.float32)
        return 0
    lax.fori_loop(0, 16, _z, 0)

    def _zs(j, _):
        pltpu.async_copy(zrows, acc_sh.at[pl.ds(s * NPE + j * 16, 16)], ss0)
        return 0
    lax.fori_loop(0, NPE // 16, _zs, 0)

    def _zw(j, _):
        pltpu.make_async_copy(
            zrows, acc_sh.at[pl.ds(s * NPE + j * 16, 16)], ss0).wait()
        return 0
    lax.fori_loop(0, NPE // 16, _zw, 0)
    plsc.subcore_barrier()

    def _scale(buf, ci):
        cvec = jnp.full((16,), ci, jnp.int32)

        def _row(e4, _):
            e = e4 * 4
            nbs = [
                plsc.load_gather(nrm2d, [cvec, jnp.full((16,), e + r, jnp.int32)])
                for r in range(4)
            ]
            for r in range(4):
                for j in range(D // 16):
                    sl = pl.ds(j * 16, 16)
                    buf[e + r, sl] = buf[e + r, sl] * nbs[r]
            return 0
        lax.fori_loop(0, CH // 4, _row, 0)

    def _group(g, _):
        pltpu.sync_copy(src_hbm.at[wid, pl.ds(g * GC, GC)], src2d)
        pltpu.sync_copy(dst_hbm.at[wid, pl.ds(g * GC, GC)], dst2d)
        pltpu.sync_copy(norm_hbm.at[wid, pl.ds(g * GC, GC)], nrm2d)
        pltpu.async_copy(t_hbm.at[src2d.at[0]], rows0, gs0)

        def _pair(p, _):
            i0 = 2 * p
            i1 = i0 + 1

            @pl.when(p > 0)
            def _():
                pltpu.make_async_copy(
                    rows1, acc_sh.at[dst2d.at[i0 - 1]], ss1).wait()
            pltpu.async_copy(t_hbm.at[src2d.at[i1]], rows1, gs1)
            pltpu.make_async_copy(t_hbm.at[src2d.at[i0]], rows0, gs0).wait()
            _scale(rows0, i0)
            pltpu.async_copy(rows0, acc_sh.at[dst2d.at[i0]], ss0, add=True)
            pltpu.make_async_copy(t_hbm.at[src2d.at[i1]], rows1, gs1).wait()
            _scale(rows1, i1)
            pltpu.async_copy(rows1, acc_sh.at[dst2d.at[i1]], ss1, add=True)

            @pl.when(p < GC // 2 - 1)
            def _():
                pltpu.make_async_copy(
                    rows0, acc_sh.at[dst2d.at[i0]], ss0).wait()
                pltpu.async_copy(t_hbm.at[src2d.at[i0 + 2]], rows0, gs0)
            return 0
        lax.fori_loop(0, GC // 2, _pair, 0)
        pltpu.make_async_copy(rows0, acc_sh.at[dst2d.at[GC - 2]], ss0).wait()
        pltpu.make_async_copy(rows1, acc_sh.at[dst2d.at[GC - 1]], ss1).wait()
        return 0
    lax.fori_loop(0, NCH // GC, _group, 0)
    plsc.subcore_barrier()

    pltpu.sync_copy(acc_sh.at[pl.ds(s * NPE, NPE)],
                    part_hbm.at[c, pl.ds(s * NPE, NPE)])


BN = 2048


def _c1_body(p_ref, h_ref, w_ref, tx1_ref, part_ref):
    tx1 = p_ref[0] + p_ref[1]
    tx1_ref[...] = tx1
    h = h_ref[...]
    part_ref[...] = (
        jnp.dot(h, w_ref[0], preferred_element_type=jnp.float32)
        + jnp.dot(tx1, w_ref[1], preferred_element_type=jnp.float32)
    )


_c1 = pl.pallas_call(
    _c1_body,
    grid=(N_PAD // BN,),
    in_specs=[
        pl.BlockSpec((NC, BN, D), lambda i: (0, i, 0)),
        pl.BlockSpec((BN, D), lambda i: (i, 0)),
        pl.BlockSpec((K, D, D), lambda i: (0, 0, 0)),
    ],
    out_specs=[
        pl.BlockSpec((BN, D), lambda i: (i, 0)),
        pl.BlockSpec((BN, D), lambda i: (i, 0)),
    ],
    out_shape=[
        jax.ShapeDtypeStruct((N_PAD, D), jnp.float32),
        jax.ShapeDtypeStruct((N_PAD, D), jnp.float32),
    ],
)


def _c2_body(p_ref, h_ref, part_ref, w_ref, b_ref, o_ref, *, final):
    tx2 = 2.0 * (p_ref[0] + p_ref[1]) - h_ref[...]
    logits = (
        part_ref[...]
        + jnp.dot(tx2, w_ref[2], preferred_element_type=jnp.float32)
        + b_ref[...]
    )
    if final:
        logits = jnp.maximum(logits, 0.0)
        col = lax.broadcasted_iota(jnp.int32, logits.shape, 1)
        logits = jnp.where(col < DOUT, logits, -jnp.inf)
        m = jnp.max(logits, axis=1, keepdims=True)
        z = jnp.exp(logits - m)
        lse = jnp.log(jnp.sum(z, axis=1, keepdims=True))
        o_ref[...] = logits - m - lse
    else:
        o_ref[...] = jnp.maximum(logits, 0.0)


def _make_c2(final):
    return pl.pallas_call(
        functools.partial(_c2_body, final=final),
        grid=(N_PAD // BN,),
        in_specs=[
            pl.BlockSpec((NC, BN, D), lambda i: (0, i, 0)),
            pl.BlockSpec((BN, D), lambda i: (i, 0)),
            pl.BlockSpec((BN, D), lambda i: (i, 0)),
            pl.BlockSpec((K, D, D), lambda i: (0, 0, 0)),
            pl.BlockSpec((1, D), lambda i: (0, 0)),
        ],
        out_specs=pl.BlockSpec((BN, D), lambda i: (i, 0)),
        out_shape=jax.ShapeDtypeStruct((N_PAD, D), jnp.float32),
    )


_c2_mid = _make_c2(False)
_c2_fin = _make_c2(True)


def kernel(x, edge_index, edge_attr, W1, b1, W2, b2, W3, b3):
    pad = E_PAD - E
    fill = (jnp.arange(pad, dtype=jnp.int32) % N)
    src = jnp.concatenate([edge_index[0], fill]).reshape(NW, NCH, CH)
    dst = jnp.concatenate([edge_index[1], fill]).reshape(NW, NCH, CH)
    w = jnp.concatenate(
        [edge_attr, jnp.zeros((pad,), jnp.float32)]).reshape(NW, NCH, CH)

    degs = _deg_kernel(src, dst, w)
    dis = _dis(degs.reshape(NC, N_PAD // D, D)).reshape(N_PAD)
    norm = _norm_kernel(dis, src, dst, w)

    xp = jnp.zeros((N_PAD, D), jnp.float32).at[:N].set(x)
    W3p = jnp.zeros((K, D, D), jnp.float32).at[:, :, :DOUT].set(W3)
    b3p = jnp.zeros((D,), jnp.float32).at[:DOUT].set(b3)

    h = xp
    for (W, b, c2) in (
        (W1, b1, _c2_mid),
        (W2, b2, _c2_mid),
        (W3p, b3p, _c2_fin),
    ):
        p1 = _prop_kernel(h, src, dst, norm)
        tx1, part = _c1(p1, h, W)
        p2 = _prop_kernel(tx1, src, dst, norm)
        h = c2(p2, h, part, W, b.reshape(1, D))

    return h[:N, :DOUT]


# speedup vs baseline: 1.3864x; 1.0502x over previous
import functools

import jax
import jax.numpy as jnp
from jax import lax
from jax.experimental import pallas as pl
from jax.experimental.pallas import tpu as pltpu
from jax.experimental.pallas import tpu_sc as plsc

N = 10000
D = 128
DOUT = 40
K = 3
NC = 2
NS = 16
NW = NC * NS
CH = 128
NPT = 320
N_PAD = NW * NPT
NPE = N_PAD // NS
E = 320000
NCH = 80
GC = 40
EPW = NCH * CH
E_PAD = NW * EPW

_MESH = plsc.VectorSubcoreMesh(core_axis_name="c", subcore_axis_name="s")
_SC_PARAMS = pltpu.CompilerParams(needs_layout_passes=False)


@functools.partial(
    pl.kernel,
    mesh=_MESH,
    compiler_params=_SC_PARAMS,
    out_type=jax.ShapeDtypeStruct((NC * N_PAD,), jnp.float32),
    scratch_types=[
        pltpu.VMEM((NCH, CH), jnp.int32),
        pltpu.VMEM((NCH, CH), jnp.int32),
        pltpu.VMEM((NCH, CH), jnp.float32),
        pltpu.VMEM((CH,), jnp.float32),
        pltpu.VMEM((NPE,), jnp.float32),
        pltpu.VMEM_SHARED((N_PAD,), jnp.float32),
    ],
)
def _deg_kernel(src_hbm, dst_hbm, w_hbm, deg_hbm,
                src2d, dst2d, w2d, wmbuf, zb, deg_sh):
    c = lax.axis_index("c")
    s = lax.axis_index("s")
    wid = s * NC + c

    def _z(i, _):
        zb[pl.ds(i * 16, 16)] = jnp.zeros((16,), jnp.float32)
        return 0
    lax.fori_loop(0, NPE // 16, _z, 0)
    pltpu.sync_copy(zb, deg_sh.at[pl.ds(s * NPE, NPE)])
    plsc.subcore_barrier()

    pltpu.sync_copy(src_hbm.at[wid], src2d)
    pltpu.sync_copy(dst_hbm.at[wid], dst2d)
    pltpu.sync_copy(w_hbm.at[wid], w2d)

    def _chunk(i, _):
        def _g(g, _):
            sl = pl.ds(g * 16, 16)
            s16 = src2d[i, sl]
            d16 = dst2d[i, sl]
            w16 = w2d[i, sl]
            wmbuf[sl] = jnp.where(s16 != d16, w16, 0.0)
            return 0
        lax.fori_loop(0, CH // 16, _g, 0)
        pltpu.sync_copy(wmbuf, deg_sh.at[src2d.at[i]], add=True)
        return 0
    lax.fori_loop(0, NCH, _chunk, 0)
    plsc.subcore_barrier()

    pltpu.sync_copy(deg_sh.at[pl.ds(s * NPE, NPE)],
                    deg_hbm.at[pl.ds(c * N_PAD + s * NPE, NPE)])


def _dis_body(deg_ref, dis_ref):
    d = deg_ref[0] + deg_ref[1]
    dis_ref[...] = jnp.where(d > 0.0, lax.rsqrt(jnp.maximum(d, 1e-30)), 0.0)


_dis = pl.pallas_call(
    _dis_body,
    out_shape=jax.ShapeDtypeStruct((N_PAD // D, D), jnp.float32),
)


@functools.partial(
    pl.kernel,
    mesh=_MESH,
    compiler_params=_SC_PARAMS,
    out_type=jax.ShapeDtypeStruct((NW, NCH, CH), jnp.float32),
    scratch_types=[
        pltpu.VMEM((NCH, CH), jnp.int32),
        pltpu.VMEM((NCH, CH), jnp.int32),
        pltpu.VMEM((NCH, CH), jnp.float32),
        pltpu.VMEM((NCH, CH), jnp.float32),
        pltpu.VMEM((N_PAD,), jnp.float32),
    ],
)
def _norm_kernel(dis_hbm, src_hbm, dst_hbm, w_hbm, norm_hbm,
                 src2d, dst2d, w2d, nrm2d, dis_v):
    c = lax.axis_index("c")
    s = lax.axis_index("s")
    wid = s * NC + c

    pltpu.sync_copy(dis_hbm, dis_v)
    pltpu.sync_copy(src_hbm.at[wid], src2d)
    pltpu.sync_copy(dst_hbm.at[wid], dst2d)
    pltpu.sync_copy(w_hbm.at[wid], w2d)

    def _chunk(i, _):
        def _g(g, _):
            sl = pl.ds(g * 16, 16)
            s16 = src2d[i, sl]
            d16 = dst2d[i, sl]
            w16 = w2d[i, sl]
            a = plsc.load_gather(dis_v, [s16])
            b = plsc.load_gather(dis_v, [d16])
            wm = jnp.where(s16 != d16, w16, 0.0)
            nrm2d[i, sl] = -(a * wm * b)
            return 0
        lax.fori_loop(0, CH // 16, _g, 0)
        return 0
    lax.fori_loop(0, NCH, _chunk, 0)
    pltpu.sync_copy(nrm2d, norm_hbm.at[wid])


@functools.partial(
    pl.kernel,
    mesh=_MESH,
    compiler_params=_SC_PARAMS,
    out_type=jax.ShapeDtypeStruct((NC, N_PAD, D), jnp.float32),
    scratch_types=[
        pltpu.VMEM((GC, CH), jnp.int32),
        pltpu.VMEM((GC, CH), jnp.int32),
        pltpu.VMEM((GC, CH), jnp.float32),
        pltpu.VMEM((CH, D), jnp.float32),
        pltpu.VMEM((CH, D), jnp.float32),
        pltpu.VMEM_SHARED((N_PAD, D), jnp.float32),
        pltpu.SemaphoreType.DMA,
        pltpu.SemaphoreType.DMA,
        pltpu.SemaphoreType.DMA,
        pltpu.SemaphoreType.DMA,
    ],
)
def _prop_kernel(t_hbm, src_hbm, dst_hbm, norm_hbm, part_hbm,
                 src2d, dst2d, nrm2d, rows0, rows1, acc_sh,
                 gs0, gs1, ss0, ss1):
    c = lax.axis_index("c")
    s = lax.axis_index("s")
    wid = s * NC + c

    def _z(i, _):
        for j in range(D // 16):
            rows0[i, pl.ds(j * 16, 16)] = jnp.zeros((16,), jnp.float32)
        return 0
    lax.fori_loop(0, 16, _z, 0)
    zsrc = rows0.at[pl.ds(0, 16)]

    def _zs(j, _):
        pltpu.async_copy(zsrc, acc_sh.at[pl.ds(s * NPE + j * 16, 16)], ss0)
        return 0
    lax.fori_loop(0, NPE // 16, _zs, 0)

    def _zw(j, _):
        pltpu.make_async_copy(
            zsrc, acc_sh.at[pl.ds(s * NPE + j * 16, 16)], ss0).wait()
        return 0
    lax.fori_loop(0, NPE // 16, _zw, 0)
    plsc.subcore_barrier()

    def _scale(buf, ci):
        cvec = jnp.full((16,), ci, jnp.int32)

        def _row(e4, _):
            e = e4 * 4
            nbs = [
                plsc.load_gather(nrm2d, [cvec, jnp.full((16,), e + r, jnp.int32)])
                for r in range(4)
            ]
            for r in range(4):
                for j in range(D // 16):
                    sl = pl.ds(j * 16, 16)
                    buf[e + r, sl] = buf[e + r, sl] * nbs[r]
            return 0
        lax.fori_loop(0, CH // 4, _row, 0)

    def _group(g, _):
        pltpu.sync_copy(src_hbm.at[wid, pl.ds(g * GC, GC)], src2d)
        pltpu.sync_copy(dst_hbm.at[wid, pl.ds(g * GC, GC)], dst2d)
        pltpu.sync_copy(norm_hbm.at[wid, pl.ds(g * GC, GC)], nrm2d)
        pltpu.async_copy(t_hbm.at[src2d.at[0]], rows0, gs0)

        def _pair(p, _):
            i0 = 2 * p
            i1 = i0 + 1

            @pl.when(p > 0)
            def _():
                pltpu.make_async_copy(
                    rows1, acc_sh.at[dst2d.at[i0 - 1]], ss1).wait()
            pltpu.async_copy(t_hbm.at[src2d.at[i1]], rows1, gs1)
            pltpu.make_async_copy(t_hbm.at[src2d.at[i0]], rows0, gs0).wait()
            _scale(rows0, i0)
            pltpu.async_copy(rows0, acc_sh.at[dst2d.at[i0]], ss0, add=True)
            pltpu.make_async_copy(t_hbm.at[src2d.at[i1]], rows1, gs1).wait()
            _scale(rows1, i1)
            pltpu.async_copy(rows1, acc_sh.at[dst2d.at[i1]], ss1, add=True)

            @pl.when(p < GC // 2 - 1)
            def _():
                pltpu.make_async_copy(
                    rows0, acc_sh.at[dst2d.at[i0]], ss0).wait()
                pltpu.async_copy(t_hbm.at[src2d.at[i0 + 2]], rows0, gs0)
            return 0
        lax.fori_loop(0, GC // 2, _pair, 0)
        pltpu.make_async_copy(rows0, acc_sh.at[dst2d.at[GC - 2]], ss0).wait()
        pltpu.make_async_copy(rows1, acc_sh.at[dst2d.at[GC - 1]], ss1).wait()
        return 0
    lax.fori_loop(0, NCH // GC, _group, 0)
    plsc.subcore_barrier()

    pltpu.sync_copy(acc_sh.at[pl.ds(s * NPE, NPE)],
                    part_hbm.at[c, pl.ds(s * NPE, NPE)])


BN = 2048


def _c1_body(p_ref, h_ref, w_ref, tx1_ref, part_ref):
    tx1 = p_ref[0] + p_ref[1]
    tx1_ref[...] = tx1
    h = h_ref[...]
    part_ref[...] = (
        jnp.dot(h, w_ref[0], preferred_element_type=jnp.float32)
        + jnp.dot(tx1, w_ref[1], preferred_element_type=jnp.float32)
    )


_c1 = pl.pallas_call(
    _c1_body,
    grid=(N_PAD // BN,),
    in_specs=[
        pl.BlockSpec((NC, BN, D), lambda i: (0, i, 0)),
        pl.BlockSpec((BN, D), lambda i: (i, 0)),
        pl.BlockSpec((K, D, D), lambda i: (0, 0, 0)),
    ],
    out_specs=[
        pl.BlockSpec((BN, D), lambda i: (i, 0)),
        pl.BlockSpec((BN, D), lambda i: (i, 0)),
    ],
    out_shape=[
        jax.ShapeDtypeStruct((N_PAD, D), jnp.float32),
        jax.ShapeDtypeStruct((N_PAD, D), jnp.float32),
    ],
)


def _c2_body(p_ref, h_ref, part_ref, w_ref, b_ref, o_ref, *, final):
    tx2 = 2.0 * (p_ref[0] + p_ref[1]) - h_ref[...]
    logits = (
        part_ref[...]
        + jnp.dot(tx2, w_ref[2], preferred_element_type=jnp.float32)
        + b_ref[...]
    )
    if final:
        logits = jnp.maximum(logits, 0.0)
        col = lax.broadcasted_iota(jnp.int32, logits.shape, 1)
        logits = jnp.where(col < DOUT, logits, -jnp.inf)
        m = jnp.max(logits, axis=1, keepdims=True)
        z = jnp.exp(logits - m)
        lse = jnp.log(jnp.sum(z, axis=1, keepdims=True))
        o_ref[...] = logits - m - lse
    else:
        o_ref[...] = jnp.maximum(logits, 0.0)


def _make_c2(final):
    return pl.pallas_call(
        functools.partial(_c2_body, final=final),
        grid=(N_PAD // BN,),
        in_specs=[
            pl.BlockSpec((NC, BN, D), lambda i: (0, i, 0)),
            pl.BlockSpec((BN, D), lambda i: (i, 0)),
            pl.BlockSpec((BN, D), lambda i: (i, 0)),
            pl.BlockSpec((K, D, D), lambda i: (0, 0, 0)),
            pl.BlockSpec((1, D), lambda i: (0, 0)),
        ],
        out_specs=pl.BlockSpec((BN, D), lambda i: (i, 0)),
        out_shape=jax.ShapeDtypeStruct((N_PAD, D), jnp.float32),
    )


_c2_mid = _make_c2(False)
_c2_fin = _make_c2(True)


def kernel(x, edge_index, edge_attr, W1, b1, W2, b2, W3, b3):
    pad = E_PAD - E
    fill = (jnp.arange(pad, dtype=jnp.int32) % N)
    src = jnp.concatenate([edge_index[0], fill]).reshape(NW, NCH, CH)
    dst = jnp.concatenate([edge_index[1], fill]).reshape(NW, NCH, CH)
    w = jnp.concatenate(
        [edge_attr, jnp.zeros((pad,), jnp.float32)]).reshape(NW, NCH, CH)

    degs = _deg_kernel(src, dst, w)
    dis = _dis(degs.reshape(NC, N_PAD // D, D)).reshape(N_PAD)
    norm = _norm_kernel(dis, src, dst, w)

    xp = jnp.zeros((N_PAD, D), jnp.float32).at[:N].set(x)
    W3p = jnp.zeros((K, D, D), jnp.float32).at[:, :, :DOUT].set(W3)
    b3p = jnp.zeros((D,), jnp.float32).at[:DOUT].set(b3)

    h = xp
    for (W, b, c2) in (
        (W1, b1, _c2_mid),
        (W2, b2, _c2_mid),
        (W3p, b3p, _c2_fin),
    ):
        p1 = _prop_kernel(h, src, dst, norm)
        tx1, part = _c1(p1, h, W)
        p2 = _prop_kernel(tx1, src, dst, norm)
        h = c2(p2, h, part, W, b.reshape(1, D))

    return h[:N, :DOUT]


# speedup vs baseline: 1.3940x; 1.0055x over previous
import functools

import jax
import jax.numpy as jnp
from jax import lax
from jax.experimental import pallas as pl
from jax.experimental.pallas import tpu as pltpu
from jax.experimental.pallas import tpu_sc as plsc

N = 10000
D = 128
DOUT = 40
K = 3
NC = 2
NS = 16
NW = NC * NS
CH = 128
NPT = 320
N_PAD = NW * NPT
NPE = N_PAD // NS
E = 320000
NCH = 80
GC = 40
EPW = NCH * CH
E_PAD = NW * EPW

_MESH = plsc.VectorSubcoreMesh(core_axis_name="c", subcore_axis_name="s")
_SC_PARAMS = pltpu.CompilerParams(needs_layout_passes=False)


@functools.partial(
    pl.kernel,
    mesh=_MESH,
    compiler_params=_SC_PARAMS,
    out_type=jax.ShapeDtypeStruct((NC * N_PAD,), jnp.float32),
    scratch_types=[
        pltpu.VMEM((NCH, CH), jnp.int32),
        pltpu.VMEM((NCH, CH), jnp.int32),
        pltpu.VMEM((NCH, CH), jnp.float32),
        pltpu.VMEM((CH,), jnp.float32),
        pltpu.VMEM((CH,), jnp.float32),
        pltpu.VMEM((NPE,), jnp.float32),
        pltpu.VMEM_SHARED((N_PAD,), jnp.float32),
        pltpu.SemaphoreType.DMA,
        pltpu.SemaphoreType.DMA,
    ],
)
def _deg_kernel(src_hbm, dst_hbm, w_hbm, deg_hbm,
                src2d, dst2d, w2d, wm0, wm1, zb, deg_sh, ds0, ds1):
    c = lax.axis_index("c")
    s = lax.axis_index("s")
    wid = s * NC + c

    def _z(i, _):
        zb[pl.ds(i * 16, 16)] = jnp.zeros((16,), jnp.float32)
        return 0
    lax.fori_loop(0, NPE // 16, _z, 0)
    pltpu.sync_copy(zb, deg_sh.at[pl.ds(s * NPE, NPE)])
    plsc.subcore_barrier()

    pltpu.sync_copy(src_hbm.at[wid], src2d)
    pltpu.sync_copy(dst_hbm.at[wid], dst2d)
    pltpu.sync_copy(w_hbm.at[wid], w2d)

    def _wm(buf, i):
        def _g(g, _):
            sl = pl.ds(g * 16, 16)
            s16 = src2d[i, sl]
            d16 = dst2d[i, sl]
            w16 = w2d[i, sl]
            buf[sl] = jnp.where(s16 != d16, w16, 0.0)
            return 0
        lax.fori_loop(0, CH // 16, _g, 0)

    def _chunkpair(p, _):
        i0 = 2 * p
        i1 = i0 + 1

        @pl.when(p > 0)
        def _():
            pltpu.make_async_copy(wm0, deg_sh.at[src2d.at[i0 - 2]], ds0).wait()
        _wm(wm0, i0)
        pltpu.async_copy(wm0, deg_sh.at[src2d.at[i0]], ds0, add=True)

        @pl.when(p > 0)
        def _():
            pltpu.make_async_copy(wm1, deg_sh.at[src2d.at[i1 - 2]], ds1).wait()
        _wm(wm1, i1)
        pltpu.async_copy(wm1, deg_sh.at[src2d.at[i1]], ds1, add=True)
        return 0
    lax.fori_loop(0, NCH // 2, _chunkpair, 0)
    pltpu.make_async_copy(wm0, deg_sh.at[src2d.at[NCH - 2]], ds0).wait()
    pltpu.make_async_copy(wm1, deg_sh.at[src2d.at[NCH - 1]], ds1).wait()
    plsc.subcore_barrier()

    pltpu.sync_copy(deg_sh.at[pl.ds(s * NPE, NPE)],
                    deg_hbm.at[pl.ds(c * N_PAD + s * NPE, NPE)])


def _dis_body(deg_ref, dis_ref):
    d = deg_ref[0] + deg_ref[1]
    dis_ref[...] = jnp.where(d > 0.0, lax.rsqrt(jnp.maximum(d, 1e-30)), 0.0)


_dis = pl.pallas_call(
    _dis_body,
    out_shape=jax.ShapeDtypeStruct((N_PAD // D, D), jnp.float32),
)


@functools.partial(
    pl.kernel,
    mesh=_MESH,
    compiler_params=_SC_PARAMS,
    out_type=jax.ShapeDtypeStruct((NW, NCH, CH), jnp.float32),
    scratch_types=[
        pltpu.VMEM((NCH, CH), jnp.int32),
        pltpu.VMEM((NCH, CH), jnp.int32),
        pltpu.VMEM((NCH, CH), jnp.float32),
        pltpu.VMEM((NCH, CH), jnp.float32),
        pltpu.VMEM((N_PAD,), jnp.float32),
    ],
)
def _norm_kernel(dis_hbm, src_hbm, dst_hbm, w_hbm, norm_hbm,
                 src2d, dst2d, w2d, nrm2d, dis_v):
    c = lax.axis_index("c")
    s = lax.axis_index("s")
    wid = s * NC + c

    pltpu.sync_copy(dis_hbm, dis_v)
    pltpu.sync_copy(src_hbm.at[wid], src2d)
    pltpu.sync_copy(dst_hbm.at[wid], dst2d)
    pltpu.sync_copy(w_hbm.at[wid], w2d)

    def _chunk(i, _):
        def _g(g, _):
            sl = pl.ds(g * 16, 16)
            s16 = src2d[i, sl]
            d16 = dst2d[i, sl]
            w16 = w2d[i, sl]
            a = plsc.load_gather(dis_v, [s16])
            b = plsc.load_gather(dis_v, [d16])
            wm = jnp.where(s16 != d16, w16, 0.0)
            nrm2d[i, sl] = -(a * wm * b)
            return 0
        lax.fori_loop(0, CH // 16, _g, 0)
        return 0
    lax.fori_loop(0, NCH, _chunk, 0)
    pltpu.sync_copy(nrm2d, norm_hbm.at[wid])


@functools.partial(
    pl.kernel,
    mesh=_MESH,
    compiler_params=_SC_PARAMS,
    out_type=jax.ShapeDtypeStruct((NC, N_PAD, D), jnp.float32),
    scratch_types=[
        pltpu.VMEM((GC, CH), jnp.int32),
        pltpu.VMEM((GC, CH), jnp.int32),
        pltpu.VMEM((GC, CH), jnp.float32),
        pltpu.VMEM((CH, D), jnp.float32),
        pltpu.VMEM((CH, D), jnp.float32),
        pltpu.VMEM_SHARED((N_PAD, D), jnp.float32),
        pltpu.SemaphoreType.DMA,
        pltpu.SemaphoreType.DMA,
        pltpu.SemaphoreType.DMA,
        pltpu.SemaphoreType.DMA,
    ],
)
def _prop_kernel(t_hbm, src_hbm, dst_hbm, norm_hbm, part_hbm,
                 src2d, dst2d, nrm2d, rows0, rows1, acc_sh,
                 gs0, gs1, ss0, ss1):
    c = lax.axis_index("c")
    s = lax.axis_index("s")
    wid = s * NC + c

    def _z(i, _):
        for j in range(D // 16):
            rows0[i, pl.ds(j * 16, 16)] = jnp.zeros((16,), jnp.float32)
        return 0
    lax.fori_loop(0, 16, _z, 0)
    zsrc = rows0.at[pl.ds(0, 16)]

    def _zs(j, _):
        pltpu.async_copy(zsrc, acc_sh.at[pl.ds(s * NPE + j * 16, 16)], ss0)
        return 0
    lax.fori_loop(0, NPE // 16, _zs, 0)

    def _zw(j, _):
        pltpu.make_async_copy(
            zsrc, acc_sh.at[pl.ds(s * NPE + j * 16, 16)], ss0).wait()
        return 0
    lax.fori_loop(0, NPE // 16, _zw, 0)
    plsc.subcore_barrier()

    def _scale(buf, ci):
        cvec = jnp.full((16,), ci, jnp.int32)

        def _row(e4, _):
            e = e4 * 4
            nbs = [
                plsc.load_gather(nrm2d, [cvec, jnp.full((16,), e + r, jnp.int32)])
                for r in range(4)
            ]
            for r in range(4):
                for j in range(D // 16):
                    sl = pl.ds(j * 16, 16)
                    buf[e + r, sl] = buf[e + r, sl] * nbs[r]
            return 0
        lax.fori_loop(0, CH // 4, _row, 0)

    def _group(g, _):
        pltpu.sync_copy(src_hbm.at[wid, pl.ds(g * GC, GC)], src2d)
        pltpu.sync_copy(dst_hbm.at[wid, pl.ds(g * GC, GC)], dst2d)
        pltpu.sync_copy(norm_hbm.at[wid, pl.ds(g * GC, GC)], nrm2d)
        pltpu.async_copy(t_hbm.at[src2d.at[0]], rows0, gs0)

        def _pair(p, _):
            i0 = 2 * p
            i1 = i0 + 1

            @pl.when(p > 0)
            def _():
                pltpu.make_async_copy(
                    rows1, acc_sh.at[dst2d.at[i0 - 1]], ss1).wait()
            pltpu.async_copy(t_hbm.at[src2d.at[i1]], rows1, gs1)
            pltpu.make_async_copy(t_hbm.at[src2d.at[i0]], rows0, gs0).wait()
            _scale(rows0, i0)
            pltpu.async_copy(rows0, acc_sh.at[dst2d.at[i0]], ss0, add=True)
            pltpu.make_async_copy(t_hbm.at[src2d.at[i1]], rows1, gs1).wait()
            _scale(rows1, i1)
            pltpu.async_copy(rows1, acc_sh.at[dst2d.at[i1]], ss1, add=True)

            @pl.when(p < GC // 2 - 1)
            def _():
                pltpu.make_async_copy(
                    rows0, acc_sh.at[dst2d.at[i0]], ss0).wait()
                pltpu.async_copy(t_hbm.at[src2d.at[i0 + 2]], rows0, gs0)
            return 0
        lax.fori_loop(0, GC // 2, _pair, 0)
        pltpu.make_async_copy(rows0, acc_sh.at[dst2d.at[GC - 2]], ss0).wait()
        pltpu.make_async_copy(rows1, acc_sh.at[dst2d.at[GC - 1]], ss1).wait()
        return 0
    lax.fori_loop(0, NCH // GC, _group, 0)
    plsc.subcore_barrier()

    pltpu.sync_copy(acc_sh.at[pl.ds(s * NPE, NPE)],
                    part_hbm.at[c, pl.ds(s * NPE, NPE)])


BN = 2048


def _c1_body(p_ref, h_ref, w_ref, tx1_ref, part_ref):
    tx1 = p_ref[0] + p_ref[1]
    tx1_ref[...] = tx1
    h = h_ref[...]
    part_ref[...] = (
        jnp.dot(h, w_ref[0], preferred_element_type=jnp.float32)
        + jnp.dot(tx1, w_ref[1], preferred_element_type=jnp.float32)
    )


_c1 = pl.pallas_call(
    _c1_body,
    grid=(N_PAD // BN,),
    in_specs=[
        pl.BlockSpec((NC, BN, D), lambda i: (0, i, 0)),
        pl.BlockSpec((BN, D), lambda i: (i, 0)),
        pl.BlockSpec((K, D, D), lambda i: (0, 0, 0)),
    ],
    out_specs=[
        pl.BlockSpec((BN, D), lambda i: (i, 0)),
        pl.BlockSpec((BN, D), lambda i: (i, 0)),
    ],
    out_shape=[
        jax.ShapeDtypeStruct((N_PAD, D), jnp.float32),
        jax.ShapeDtypeStruct((N_PAD, D), jnp.float32),
    ],
)


def _c2_body(p_ref, h_ref, part_ref, w_ref, b_ref, o_ref, *, final):
    tx2 = 2.0 * (p_ref[0] + p_ref[1]) - h_ref[...]
    logits = (
        part_ref[...]
        + jnp.dot(tx2, w_ref[2], preferred_element_type=jnp.float32)
        + b_ref[...]
    )
    if final:
        logits = jnp.maximum(logits, 0.0)
        col = lax.broadcasted_iota(jnp.int32, logits.shape, 1)
        logits = jnp.where(col < DOUT, logits, -jnp.inf)
        m = jnp.max(logits, axis=1, keepdims=True)
        z = jnp.exp(logits - m)
        lse = jnp.log(jnp.sum(z, axis=1, keepdims=True))
        o_ref[...] = logits - m - lse
    else:
        o_ref[...] = jnp.maximum(logits, 0.0)


def _make_c2(final):
    return pl.pallas_call(
        functools.partial(_c2_body, final=final),
        grid=(N_PAD // BN,),
        in_specs=[
            pl.BlockSpec((NC, BN, D), lambda i: (0, i, 0)),
            pl.BlockSpec((BN, D), lambda i: (i, 0)),
            pl.BlockSpec((BN, D), lambda i: (i, 0)),
            pl.BlockSpec((K, D, D), lambda i: (0, 0, 0)),
            pl.BlockSpec((1, D), lambda i: (0, 0)),
        ],
        out_specs=pl.BlockSpec((BN, D), lambda i: (i, 0)),
        out_shape=jax.ShapeDtypeStruct((N_PAD, D), jnp.float32),
    )


_c2_mid = _make_c2(False)
_c2_fin = _make_c2(True)


def kernel(x, edge_index, edge_attr, W1, b1, W2, b2, W3, b3):
    pad = E_PAD - E
    fill = (jnp.arange(pad, dtype=jnp.int32) % N)
    src = jnp.concatenate([edge_index[0], fill]).reshape(NW, NCH, CH)
    dst = jnp.concatenate([edge_index[1], fill]).reshape(NW, NCH, CH)
    w = jnp.concatenate(
        [edge_attr, jnp.zeros((pad,), jnp.float32)]).reshape(NW, NCH, CH)

    degs = _deg_kernel(src, dst, w)
    dis = _dis(degs.reshape(NC, N_PAD // D, D)).reshape(N_PAD)
    norm = _norm_kernel(dis, src, dst, w)

    xp = jnp.zeros((N_PAD, D), jnp.float32).at[:N].set(x)
    W3p = jnp.zeros((K, D, D), jnp.float32).at[:, :, :DOUT].set(W3)
    b3p = jnp.zeros((D,), jnp.float32).at[:DOUT].set(b3)

    h = xp
    for (W, b, c2) in (
        (W1, b1, _c2_mid),
        (W2, b2, _c2_mid),
        (W3p, b3p, _c2_fin),
    ):
        p1 = _prop_kernel(h, src, dst, norm)
        tx1, part = _c1(p1, h, W)
        p2 = _prop_kernel(tx1, src, dst, norm)
        h = c2(p2, h, part, W, b.reshape(1, D))

    return h[:N, :DOUT]
